# baseline (speedup 1.0000x reference)
import functools

import jax
import jax.numpy as jnp
from jax import lax
from jax.experimental import pallas as pl
from jax.experimental.pallas import tpu as pltpu
from jax.experimental.pallas import tpu_sc as plsc

D = 32
B = 16384
NC = 2
NS = 16
NW = NC * NS
B_PER_W = B // NW
CHUNK = 128
NCHUNK = B_PER_W // CHUNK

_mesh = plsc.VectorSubcoreMesh(core_axis_name="c", subcore_axis_name="s")


@functools.partial(
    pl.kernel,
    mesh=_mesh,
    out_type=jax.ShapeDtypeStruct((NW, NCHUNK, CHUNK, D), jnp.float32),
    scratch_types=[
        pltpu.VMEM((NCHUNK, CHUNK), jnp.int32),
        pltpu.VMEM((NCHUNK, CHUNK, D), jnp.float32),
        pltpu.SemaphoreType.DMA,
    ],
    compiler_params=pltpu.CompilerParams(use_tc_tiling_on_sc=False),
)
def _sc_gather(idx_hbm, table_hbm, out_hbm, idx_v, rows_v, sem):
    wid = lax.axis_index("s") * NC + lax.axis_index("c")
    pltpu.sync_copy(idx_hbm.at[wid], idx_v)
    copies = []
    for j in range(NCHUNK):
        copies.append(
            pltpu.async_copy(table_hbm.at[idx_v.at[j]], rows_v.at[j], sem)
        )
    for c in copies:
        c.wait()
    pltpu.sync_copy(rows_v, out_hbm.at[wid])


def kernel(storm_names, storm_embed_weight):
    idx = storm_names.astype(jnp.int32).reshape(NW, NCHUNK, CHUNK)
    out = _sc_gather(idx, storm_embed_weight)
    return out.reshape(B, D)


# speedup vs baseline: 1.5810x; 1.5810x over previous
import functools

import jax
import jax.numpy as jnp
from jax import lax
from jax.experimental import pallas as pl
from jax.experimental.pallas import tpu as pltpu
from jax.experimental.pallas import tpu_sc as plsc

D = 32
B = 16384
V = 1000000
NC = 2
NS = 16
NW = NC * NS
B_PER_W = B // NW
CHUNK = 128
NCHUNK = B_PER_W // CHUNK
KWIN = 32
L = 16

_mesh = plsc.VectorSubcoreMesh(core_axis_name="c", subcore_axis_name="s")


@functools.partial(
    pl.kernel,
    mesh=_mesh,
    out_type=jax.ShapeDtypeStruct((B * D,), jnp.float32),
    scratch_types=[
        pltpu.SMEM((B_PER_W,), jnp.int32),
        pltpu.VMEM((B_PER_W,), jnp.int32),
        pltpu.VMEM_SHARED((NW, B_PER_W), jnp.int32),
        pltpu.VMEM((B_PER_W, D), jnp.float32),
        pltpu.VMEM((B_PER_W * D,), jnp.float32),
        pltpu.SemaphoreType.DMA,
        pltpu.SemaphoreType.DMA,
    ],
    compiler_params=pltpu.CompilerParams(needs_layout_passes=False),
)
def _sc_gather(idx_hbm, table_hbm, out_hbm, idx_s, idx_v, idx_sh, rows_v,
               flat_v, gsem, osem):
    wid = lax.axis_index("s") * NC + lax.axis_index("c")
    base = wid * B_PER_W
    pltpu.sync_copy(idx_hbm.at[pl.ds(base, B_PER_W)], idx_v)
    pltpu.sync_copy(idx_v, idx_sh.at[wid])

    lanes = lax.iota(jnp.int32, L)

    def _stage_verified(carry):
        pltpu.sync_copy(idx_sh.at[wid], idx_s)

        def check_group(g, mism):
            vec = idx_v[pl.ds(g * L, L)]
            svec = jnp.zeros((L,), jnp.int32)
            for k in range(L):
                sv = jnp.full((L,), idx_s[g * L + k], jnp.int32)
                svec = jnp.where(lanes == k, sv, svec)
            return mism | lax.bitwise_xor(vec, svec)

        mism = lax.fori_loop(0, B_PER_W // L, check_group,
                             jnp.zeros((L,), jnp.int32))
        return jnp.max(mism)

    lax.while_loop(lambda m: m != 0, lambda m: _stage_verified(m),
                   _stage_verified(jnp.int32(0)))

    def issue_row(j, carry):
        i = idx_s[j]
        pltpu.async_copy(table_hbm.at[i], rows_v.at[j], gsem)
        return carry

    def drain_one(j, carry):
        pltpu.make_async_copy(table_hbm.at[0], rows_v.at[0], gsem).wait()
        return carry

    out_copies = []
    for c in range(NCHUNK):
        lo = c * CHUNK
        lax.fori_loop(lo, lo + KWIN, issue_row, 0, unroll=4)

        def steady(j, carry):
            drain_one(j, 0)
            return issue_row(j + KWIN, carry)

        lax.fori_loop(lo, lo + CHUNK - KWIN, steady, 0, unroll=4)
        lax.fori_loop(0, KWIN, drain_one, 0, unroll=4)

        def repack(j, carry):
            f = j * D
            flat_v[pl.ds(f, L)] = rows_v[j, pl.ds(0, L)]
            flat_v[pl.ds(f + L, L)] = rows_v[j, pl.ds(L, L)]
            return carry

        lax.fori_loop(lo, lo + CHUNK, repack, 0, unroll=4)

        out_copies.append(
            pltpu.async_copy(
                flat_v.at[pl.ds(lo * D, CHUNK * D)],
                out_hbm.at[pl.ds((base + lo) * D, CHUNK * D)], osem)
        )
    for oc in out_copies:
        oc.wait()


def kernel(storm_names, storm_embed_weight):
    idx = storm_names.astype(jnp.int32)
    return _sc_gather(idx, storm_embed_weight).reshape(B, D)
